# Initial kernel scaffold; baseline (speedup 1.0000x reference)
import jax
import jax.numpy as jnp
from jax.experimental import pallas as pl


def kernel(spatial_pos, spatial_embeddings):
    raise NotImplementedError("write your pallas kernel here")


# speedup vs baseline: 15.4228x; 15.4228x over previous
import functools

import jax
import jax.numpy as jnp
from jax import lax
from jax.experimental import pallas as pl
from jax.experimental.pallas import tpu as pltpu
from jax.experimental.pallas import tpu_sc as plsc

B, L, H = 8, 512, 16
V = 513
SUPER = 512
W = 520
NW = 32
Q = 4
ROWS_PER_W = L // Q
CH_ROWS = 4
CHUNK = CH_ROWS * L
N_CHUNKS = ROWS_PER_W // CH_ROWS


def _sc_body(pos_hbm, tab_hbm, out_hbm, tab_v, idx_v, out_v):
    wid = lax.axis_index("s") * 2 + lax.axis_index("c")
    b = wid // Q
    base_off = (wid % Q) * ROWS_PER_W * L

    pltpu.sync_copy(tab_hbm, tab_v)
    lane = lax.iota(jnp.int32, 16)
    sup = jnp.full((16,), SUPER, jnp.int32)

    def chunk_body(t, carry):
        off = base_off + t * CHUNK
        pltpu.sync_copy(pos_hbm.at[b, pl.ds(off, CHUNK)], idx_v)

        def vec_body(v, c):
            o = pl.multiple_of(v * 16, 16)
            idx = idx_v[pl.ds(o, 16)]
            g = off + o
            j0 = g & (L - 1)
            i_cur = g >> 9
            is_sup = ((j0 + lane) == 0) | (i_cur == 0)
            idxf = jnp.where(is_sup, sup, idx)
            for h in range(H):
                vals = plsc.load_gather(tab_v, [idxf + (h * W)])
                out_v[h, pl.ds(o, 16)] = vals
            return c

        lax.fori_loop(0, CHUNK // 16, vec_body, 0)
        for h in range(H):
            pltpu.sync_copy(out_v.at[h], out_hbm.at[b, h, pl.ds(off, CHUNK)])
        return carry

    lax.fori_loop(0, N_CHUNKS, chunk_body, 0)


def kernel(spatial_pos, spatial_embeddings):
    pos_flat = spatial_pos.reshape(B, L * L)
    tab = (
        jnp.zeros((H, W), jnp.float32)
        .at[:, :V].set(spatial_embeddings.T)
        .reshape(H * W)
    )
    mesh = plsc.VectorSubcoreMesh(
        core_axis_name="c", subcore_axis_name="s", num_cores=2, num_subcores=16
    )
    run = functools.partial(
        pl.kernel,
        out_type=jax.ShapeDtypeStruct((B, H, L * L), jnp.float32),
        mesh=mesh,
        scratch_types=[
            pltpu.VMEM((H * W,), jnp.float32),
            pltpu.VMEM((CHUNK,), jnp.int32),
            pltpu.VMEM((H, CHUNK), jnp.float32),
        ],
        compiler_params=pltpu.CompilerParams(needs_layout_passes=False),
    )(_sc_body)
    return run(pos_flat, tab).reshape(B, H, L, L)


# speedup vs baseline: 16.9885x; 1.1015x over previous
import functools

import jax
import jax.numpy as jnp
from jax import lax
from jax.experimental import pallas as pl
from jax.experimental.pallas import tpu as pltpu
from jax.experimental.pallas import tpu_sc as plsc

B, L, H = 8, 512, 16
V = 513
SUPER = 512
W = 520
NW = 32
Q = 4
ROWS_PER_W = L // Q
CH_ROWS = 4
CHUNK = CH_ROWS * L
N_CHUNKS = ROWS_PER_W // CH_ROWS


def _sc_body(pos_hbm, tab_hbm, out_hbm, tab_v, idx_v, out_v, sem_in, sem_out):
    wid = lax.axis_index("s") * 2 + lax.axis_index("c")
    b = wid // Q
    base_off = (wid % Q) * ROWS_PER_W * L

    pltpu.sync_copy(tab_hbm, tab_v)
    lane = lax.iota(jnp.int32, 16)
    sup = jnp.full((16,), SUPER, jnp.int32)

    def idx_copy(chunk, buf):
        off = base_off + chunk * CHUNK
        return pltpu.make_async_copy(
            pos_hbm.at[b, pl.ds(off, CHUNK)], idx_v.at[buf], sem_in
        )

    def out_copy(chunk, buf, h):
        off = base_off + chunk * CHUNK
        return pltpu.make_async_copy(
            out_v.at[buf, h], out_hbm.at[b, h, pl.ds(off, CHUNK)], sem_out
        )

    idx_copy(0, 0).start()
    idx_copy(1, 1).start()

    def process(t, chunk, buf):
        idx_copy(chunk, buf).wait()

        @pl.when(t > 0)
        def _drain():
            for h in range(H):
                out_copy(chunk, buf, h).wait()

        off = base_off + chunk * CHUNK

        def vec_body(v, c):
            o = pl.multiple_of(v * 16, 16)
            idx = idx_v[buf, pl.ds(o, 16)]
            g = off + o
            j0 = g & (L - 1)
            i_cur = g >> 9
            is_sup = ((j0 + lane) == 0) | (i_cur == 0)
            idxf = jnp.where(is_sup, sup, idx)
            for h in range(H):
                vals = plsc.load_gather(tab_v, [idxf + (h * W)])
                out_v[buf, h, pl.ds(o, 16)] = vals
            return c

        lax.fori_loop(0, CHUNK // 16, vec_body, 0)
        for h in range(H):
            out_copy(chunk, buf, h).start()

        @pl.when(t < N_CHUNKS // 2 - 1)
        def _prefetch():
            idx_copy(chunk + 2, buf).start()

    def pair_body(t, c):
        process(t, 2 * t, 0)
        process(t, 2 * t + 1, 1)
        return c

    lax.fori_loop(0, N_CHUNKS // 2, pair_body, 0)
    for buf in range(2):
        for h in range(H):
            out_copy(0, buf, h).wait()


def kernel(spatial_pos, spatial_embeddings):
    pos_flat = spatial_pos.reshape(B, L * L)
    tab = (
        jnp.zeros((H, W), jnp.float32)
        .at[:, :V].set(spatial_embeddings.T)
        .reshape(H * W)
    )
    mesh = plsc.VectorSubcoreMesh(
        core_axis_name="c", subcore_axis_name="s", num_cores=2, num_subcores=16
    )
    run = functools.partial(
        pl.kernel,
        out_type=jax.ShapeDtypeStruct((B, H, L * L), jnp.float32),
        mesh=mesh,
        scratch_types=[
            pltpu.VMEM((H * W,), jnp.float32),
            pltpu.VMEM((2, CHUNK), jnp.int32),
            pltpu.VMEM((2, H, CHUNK), jnp.float32),
            pltpu.SemaphoreType.DMA,
            pltpu.SemaphoreType.DMA,
        ],
        compiler_params=pltpu.CompilerParams(needs_layout_passes=False),
    )(_sc_body)
    return run(pos_flat, tab).reshape(B, H, L, L)


# speedup vs baseline: 33.6280x; 1.9795x over previous
import functools

import jax
import jax.numpy as jnp
from jax import lax
from jax.experimental import pallas as pl
from jax.experimental.pallas import tpu as pltpu
from jax.experimental.pallas import tpu_sc as plsc

B, L, H = 8, 512, 16
V = 513
SUPER = 512
W = 520
NW = 32
Q = 4
ROWS_PER_W = L // Q
CH_ROWS = 4
CHUNK = CH_ROWS * L
N_CHUNKS = ROWS_PER_W // CH_ROWS


def _sc_body(pos_hbm, tab_hbm, out_hbm, tab_v, idx_v, out_v, sem_in, sem_out):
    wid = lax.axis_index("s") * 2 + lax.axis_index("c")
    b = wid // Q
    base_off = (wid % Q) * ROWS_PER_W * L

    pltpu.sync_copy(tab_hbm, tab_v)
    lane = lax.iota(jnp.int32, 16)
    sup = jnp.full((16,), SUPER, jnp.int32)

    def idx_copy(chunk, buf):
        off = base_off + chunk * CHUNK
        return pltpu.make_async_copy(
            pos_hbm.at[b, pl.ds(off, CHUNK)], idx_v.at[buf], sem_in
        )

    def out_copy(chunk, buf, h):
        off = base_off + chunk * CHUNK
        return pltpu.make_async_copy(
            out_v.at[buf, h], out_hbm.at[b, h, pl.ds(off, CHUNK)], sem_out
        )

    idx_copy(0, 0).start()
    idx_copy(1, 1).start()

    def process(t, chunk, buf):
        idx_copy(chunk, buf).wait()

        @pl.when(t > 0)
        def _drain():
            for h in range(H):
                out_copy(chunk, buf, h).wait()

        off = base_off + chunk * CHUNK

        @plsc.parallel_loop(0, CHUNK, 16, unroll=4)
        def vec_body(o):
            o = pl.multiple_of(o, 16)
            idx = idx_v[buf, pl.ds(o, 16)]
            g = off + o
            j0 = g & (L - 1)
            i_cur = g >> 9
            is_sup = ((j0 + lane) == 0) | (i_cur == 0)
            idxf = jnp.where(is_sup, sup, idx)
            for h in range(H):
                vals = plsc.load_gather(tab_v, [idxf + (h * W)])
                out_v[buf, h, pl.ds(o, 16)] = vals
        for h in range(H):
            out_copy(chunk, buf, h).start()

        @pl.when(t < N_CHUNKS // 2 - 1)
        def _prefetch():
            idx_copy(chunk + 2, buf).start()

    def pair_body(t, c):
        process(t, 2 * t, 0)
        process(t, 2 * t + 1, 1)
        return c

    lax.fori_loop(0, N_CHUNKS // 2, pair_body, 0)
    for buf in range(2):
        for h in range(H):
            out_copy(0, buf, h).wait()


def kernel(spatial_pos, spatial_embeddings):
    pos_flat = spatial_pos.reshape(B, L * L)
    tab = (
        jnp.zeros((H, W), jnp.float32)
        .at[:, :V].set(spatial_embeddings.T)
        .reshape(H * W)
    )
    mesh = plsc.VectorSubcoreMesh(
        core_axis_name="c", subcore_axis_name="s", num_cores=2, num_subcores=16
    )
    run = functools.partial(
        pl.kernel,
        out_type=jax.ShapeDtypeStruct((B, H, L * L), jnp.float32),
        mesh=mesh,
        scratch_types=[
            pltpu.VMEM((H * W,), jnp.float32),
            pltpu.VMEM((2, CHUNK), jnp.int32),
            pltpu.VMEM((2, H, CHUNK), jnp.float32),
            pltpu.SemaphoreType.DMA,
            pltpu.SemaphoreType.DMA,
        ],
        compiler_params=pltpu.CompilerParams(needs_layout_passes=False),
    )(_sc_body)
    return run(pos_flat, tab).reshape(B, H, L, L)
